# baseline (speedup 1.0000x reference)
import jax
import jax.numpy as jnp
from jax.experimental import pallas as pl

NUM_CLASSES = 1000
ROWS = 16384 * 26
BLOCK_R = 512


def _onehot_body(x_ref, o_ref):
    cols = jax.lax.broadcasted_iota(jnp.int32, (BLOCK_R, NUM_CLASSES), 1)
    o_ref[...] = (x_ref[...] == cols).astype(jnp.float32)


def kernel(x):
    xf = x.reshape(ROWS, 1).astype(jnp.int32)
    out = pl.pallas_call(
        _onehot_body,
        grid=(ROWS // BLOCK_R,),
        in_specs=[pl.BlockSpec((BLOCK_R, 1), lambda i: (i, 0))],
        out_specs=pl.BlockSpec((BLOCK_R, NUM_CLASSES), lambda i: (i, 0)),
        out_shape=jax.ShapeDtypeStruct((ROWS, NUM_CLASSES), jnp.float32),
    )(xf)
    return out.reshape(16384, 26, NUM_CLASSES)


# speedup vs baseline: 1.5712x; 1.5712x over previous
import jax
import jax.numpy as jnp
from jax.experimental import pallas as pl

NUM_CLASSES = 1000
B0 = 64


def _onehot_body(x_ref, o_ref):
    cols = jax.lax.broadcasted_iota(jnp.int32, (B0, 26, NUM_CLASSES), 2)
    o_ref[...] = (x_ref[...][:, :, None] == cols).astype(jnp.float32)


def kernel(x):
    xi = x.astype(jnp.int32)
    return pl.pallas_call(
        _onehot_body,
        grid=(16384 // B0,),
        in_specs=[pl.BlockSpec((B0, 26), lambda i: (i, 0))],
        out_specs=pl.BlockSpec((B0, 26, NUM_CLASSES), lambda i: (i, 0, 0)),
        out_shape=jax.ShapeDtypeStruct((16384, 26, NUM_CLASSES), jnp.float32),
    )(xi)
